# baseline (speedup 1.0000x reference)
import functools

import jax
import jax.numpy as jnp
from jax import lax
from jax.experimental import pallas as pl
from jax.experimental.pallas import tpu as pltpu
from jax.experimental.pallas import tpu_sc as plsc


def _prep_body(aa_ref, pos_ref, pc_ref, pcW_ref, pcb_ref, tok_ref, sv_ref,
               ev_ref, sW1_ref, sb1_ref, sW2_ref, sb2_ref, eW1_ref, eb1_ref,
               eW2_ref, eb2_ref, nW_ref, nb_ref, TP_out, S_out, idx_out):
    hp = jax.lax.Precision.HIGHEST
    L = idx_out.shape[1]
    nW = nW_ref[...]
    nW_aa, nW_pos, nW_pc = nW[0:32, :], nW[32:48, :], nW[48:64, :]
    nW_st, nW_ev = nW[64:96, :], nW[96:128, :]

    pc_feat = (jnp.dot(pc_ref[...], pcW_ref[...], precision=hp)
               + pcb_ref[...][None, :])
    T = (jnp.dot(aa_ref[...], nW_aa, precision=hp)
         + jnp.dot(pc_feat, nW_pc, precision=hp))
    P = jnp.dot(pos_ref[0:L, :], nW_pos, precision=hp)
    V, H = T.shape
    TP_out[...] = (P[:, None, :] + T[None, :, :]).reshape(L * V, H)

    tok = tok_ref[...]
    idx_out[...] = tok + V * jax.lax.broadcasted_iota(jnp.int32, tok.shape, 1)

    sv = sv_ref[...]
    f = jnp.concatenate([
        sv[:, 0:1] * 0.1,
        sv[:, 1:2] * (1.0 / 2000.0),
        jnp.log1p(jnp.maximum(sv[:, 2:3], 0.0)) * (1.0 / 20.0),
    ], axis=1)
    f = jnp.nan_to_num(f, nan=0.0, posinf=10.0, neginf=-10.0)
    hs = jnp.maximum(
        jnp.dot(f, sW1_ref[...], precision=hp) + sb1_ref[...][None, :], 0.0)
    s32 = jnp.dot(hs, sW2_ref[...], precision=hp) + sb2_ref[...][None, :]

    e = ev_ref[...] * 0.01
    e = jnp.nan_to_num(e, nan=0.0, posinf=10.0, neginf=-10.0)
    he = jnp.maximum(
        jnp.dot(e, eW1_ref[...], precision=hp) + eb1_ref[...][None, :], 0.0)
    e32 = jnp.dot(he, eW2_ref[...], precision=hp) + eb2_ref[...][None, :]

    S_out[...] = (jnp.dot(s32, nW_st, precision=hp)
                  + jnp.dot(e32, nW_ev, precision=hp) + nb_ref[...][None, :])


def _make_sc_main(BK, L, H, NC, NS):
    NW = NC * NS
    BPW = BK // NW
    NJ = H // 16
    f32 = jnp.float32

    NR = 4

    @functools.partial(
        pl.kernel,
        out_type=jax.ShapeDtypeStruct((BK, L, H), f32),
        mesh=plsc.VectorSubcoreMesh(core_axis_name="c", subcore_axis_name="s"),
        scratch_types=[
            pltpu.VMEM((BPW, L), jnp.int32),
            pltpu.VMEM((BPW, H), f32),
            pltpu.VMEM((NR, L, H), f32),
            pltpu.VMEM((NR, L, H), f32),
            pltpu.VMEM((1, H), f32),
            pltpu.VMEM((1, H), f32),
        ] + [pltpu.SemaphoreType.DMA] * (2 * NR),
    )
    def sc_main(TP_hbm, idx_hbm, S_hbm, gam_hbm, bet_hbm, out_hbm,
                idx_v, S_v, gb, ob, gam_v, bet_v, *sems):
        sgs = sems[:NR]
        sos = sems[NR:]
        wid = lax.axis_index("s") * NC + lax.axis_index("c")
        b0 = wid * BPW
        pltpu.sync_copy(idx_hbm.at[pl.ds(b0, BPW)], idx_v)
        pltpu.sync_copy(S_hbm.at[pl.ds(b0, BPW)], S_v)
        pltpu.sync_copy(gam_hbm, gam_v)
        pltpu.sync_copy(bet_hbm, bet_v)

        gam = [gam_v[0, pl.ds(16 * j, 16)] for j in range(NJ)]
        bet = [bet_v[0, pl.ds(16 * j, 16)] for j in range(NJ)]
        lanes = lax.iota(jnp.int32, 16)
        perms = [(lanes ^ c)[:, None] for c in (8, 4, 2, 1)]
        dnums = lax.GatherDimensionNumbers(
            offset_dims=(), collapsed_slice_dims=(0,), start_index_map=(0,))

        def lane_swap(v, perm):
            return lax.gather(v, perm, dnums, slice_sizes=(1,),
                              mode=lax.GatherScatterMode.PROMISE_IN_BOUNDS)

        for r in range(NR - 1):
            pltpu.async_copy(TP_hbm.at[idx_v.at[r]], gb.at[r], sgs[r])

        def do_b(b, p):
            pltpu.make_async_copy(TP_hbm.at[idx_v.at[b]], gb.at[p],
                                  sgs[p]).wait()

            pn = (p + NR - 1) % NR

            @pl.when(b + NR - 1 < BPW)
            def _():
                pltpu.async_copy(TP_hbm.at[idx_v.at[b + NR - 1]], gb.at[pn],
                                 sgs[pn])

            @pl.when(b >= NR)
            def _():
                pltpu.make_async_copy(ob.at[p], out_hbm.at[b0 + b - NR],
                                      sos[p]).wait()

            Sb = [S_v[b, pl.ds(16 * j, 16)] for j in range(NJ)]
            gbp = gb.at[p]
            obp = ob.at[p]

            @plsc.parallel_loop(0, L, unroll=2)
            def row(l):
                x = [gbp[l, pl.ds(16 * j, 16)] + Sb[j] for j in range(NJ)]
                s = (((x[0] + x[1]) + (x[2] + x[3]))
                     + ((x[4] + x[5]) + (x[6] + x[7])))
                q = ((((x[0] * x[0] + x[1] * x[1])
                       + (x[2] * x[2] + x[3] * x[3]))
                      + ((x[4] * x[4] + x[5] * x[5])
                         + (x[6] * x[6] + x[7] * x[7]))))
                for perm in perms:
                    s = s + lane_swap(s, perm)
                    q = q + lane_swap(q, perm)
                mu = s * (1.0 / H)
                var = q * (1.0 / H) - mu * mu
                a = var + 1e-5
                ai = lax.bitcast_convert_type(a, jnp.int32)
                y = lax.bitcast_convert_type(
                    jnp.int32(0x5F375A86) - (ai >> 1), f32)
                y = y * (1.5 - 0.5 * a * y * y)
                for j in range(NJ):
                    obp[l, pl.ds(16 * j, 16)] = jnp.maximum(
                        (x[j] - mu) * y * gam[j] + bet[j], 0.0)
            pltpu.async_copy(obp, out_hbm.at[b0 + b], sos[p])

        def bodyn(i, carry):
            for r in range(NR):
                do_b(NR * i + r, r)
            return carry

        lax.fori_loop(0, BPW // NR, bodyn, 0)
        for r in range(NR):
            pltpu.make_async_copy(ob.at[r], out_hbm.at[b0 + BPW - NR + r],
                                  sos[r]).wait()

    return sc_main


def kernel(seq_tokens, state_vars, env_vars, aa_emb, pos_emb, pc_table, pc_W,
           pc_b, sW1, sb1, sW2, sb2, eW1, eb1, eW2, eb2, nW, nb, gamma, beta):
    B, L = seq_tokens.shape
    V, H = aa_emb.shape[0], nW.shape[1]
    f32 = jnp.float32

    TP, S, idx = pl.pallas_call(
        _prep_body,
        out_shape=[
            jax.ShapeDtypeStruct((L * V, H), f32),
            jax.ShapeDtypeStruct((B, H), f32),
            jax.ShapeDtypeStruct((B, L), jnp.int32),
        ],
    )(aa_emb, pos_emb, pc_table, pc_W, pc_b, seq_tokens, state_vars, env_vars,
      sW1, sb1, sW2, sb2, eW1, eb1, eW2, eb2, nW, nb)

    info = plsc.get_sparse_core_info()
    gam2 = gamma.reshape(1, -1)
    bet2 = beta.reshape(1, -1)

    NSPLIT = 2
    BK = B // NSPLIT
    sc_main = _make_sc_main(BK, L, H, info.num_cores, info.num_subcores)
    outs = [
        sc_main(TP, idx[k * BK:(k + 1) * BK], S[k * BK:(k + 1) * BK],
                gam2, bet2)
        for k in range(NSPLIT)
    ]
    return jnp.concatenate(outs, axis=0)


# speedup vs baseline: 1.3269x; 1.3269x over previous
import functools

import jax
import jax.numpy as jnp
from jax import lax
from jax.experimental import pallas as pl
from jax.experimental.pallas import tpu as pltpu
from jax.experimental.pallas import tpu_sc as plsc


def _prep_body(aa_ref, pos_ref, pc_ref, pcW_ref, pcb_ref, tok_ref, sv_ref,
               ev_ref, sW1_ref, sb1_ref, sW2_ref, sb2_ref, eW1_ref, eb1_ref,
               eW2_ref, eb2_ref, nW_ref, nb_ref, TP_out, S_out, idx_out):
    hp = jax.lax.Precision.HIGHEST
    L = idx_out.shape[1]
    nW = nW_ref[...]
    nW_aa, nW_pos, nW_pc = nW[0:32, :], nW[32:48, :], nW[48:64, :]
    nW_st, nW_ev = nW[64:96, :], nW[96:128, :]

    pc_feat = (jnp.dot(pc_ref[...], pcW_ref[...], precision=hp)
               + pcb_ref[...][None, :])
    T = (jnp.dot(aa_ref[...], nW_aa, precision=hp)
         + jnp.dot(pc_feat, nW_pc, precision=hp))
    P = jnp.dot(pos_ref[0:L, :], nW_pos, precision=hp)
    V, H = T.shape
    TP_out[...] = (P[:, None, :] + T[None, :, :]).reshape(L * V, H)

    tok = tok_ref[...]
    idx_out[...] = tok + V * jax.lax.broadcasted_iota(jnp.int32, tok.shape, 1)

    sv = sv_ref[...]
    f = jnp.concatenate([
        sv[:, 0:1] * 0.1,
        sv[:, 1:2] * (1.0 / 2000.0),
        jnp.log1p(jnp.maximum(sv[:, 2:3], 0.0)) * (1.0 / 20.0),
    ], axis=1)
    f = jnp.nan_to_num(f, nan=0.0, posinf=10.0, neginf=-10.0)
    hs = jnp.maximum(
        jnp.dot(f, sW1_ref[...], precision=hp) + sb1_ref[...][None, :], 0.0)
    s32 = jnp.dot(hs, sW2_ref[...], precision=hp) + sb2_ref[...][None, :]

    e = ev_ref[...] * 0.01
    e = jnp.nan_to_num(e, nan=0.0, posinf=10.0, neginf=-10.0)
    he = jnp.maximum(
        jnp.dot(e, eW1_ref[...], precision=hp) + eb1_ref[...][None, :], 0.0)
    e32 = jnp.dot(he, eW2_ref[...], precision=hp) + eb2_ref[...][None, :]

    S_out[...] = (jnp.dot(s32, nW_st, precision=hp)
                  + jnp.dot(e32, nW_ev, precision=hp) + nb_ref[...][None, :])


def _make_sc_main(BK, L, H, NC, NS):
    NW = NC * NS
    BPW = BK // NW
    NJ = H // 16
    f32 = jnp.float32

    NR = 4

    LPAD = (L + 7) // 8 * 8

    @functools.partial(
        pl.kernel,
        out_type=jax.ShapeDtypeStruct((BK, LPAD, H), f32),
        mesh=plsc.VectorSubcoreMesh(core_axis_name="c", subcore_axis_name="s"),
        scratch_types=[
            pltpu.VMEM((BPW, L), jnp.int32),
            pltpu.VMEM((BPW, H), f32),
            pltpu.VMEM((NR, L, H), f32),
            pltpu.VMEM((NR, LPAD, H), f32),
            pltpu.VMEM((1, H), f32),
            pltpu.VMEM((1, H), f32),
        ] + [pltpu.SemaphoreType.DMA] * (2 * NR),
    )
    def sc_main(TP_hbm, idx_hbm, S_hbm, gam_hbm, bet_hbm, out_hbm,
                idx_v, S_v, gb, ob, gam_v, bet_v, *sems):
        sgs = sems[:NR]
        sos = sems[NR:]
        wid = lax.axis_index("s") * NC + lax.axis_index("c")
        b0 = wid * BPW
        pltpu.sync_copy(idx_hbm.at[pl.ds(b0, BPW)], idx_v)
        pltpu.sync_copy(S_hbm.at[pl.ds(b0, BPW)], S_v)
        pltpu.sync_copy(gam_hbm, gam_v)
        pltpu.sync_copy(bet_hbm, bet_v)

        gam = [gam_v[0, pl.ds(16 * j, 16)] for j in range(NJ)]
        bet = [bet_v[0, pl.ds(16 * j, 16)] for j in range(NJ)]
        lanes = lax.iota(jnp.int32, 16)
        perms = [(lanes ^ c)[:, None] for c in (8, 4, 2, 1)]
        dnums = lax.GatherDimensionNumbers(
            offset_dims=(), collapsed_slice_dims=(0,), start_index_map=(0,))

        def lane_swap(v, perm):
            return lax.gather(v, perm, dnums, slice_sizes=(1,),
                              mode=lax.GatherScatterMode.PROMISE_IN_BOUNDS)

        for r in range(NR - 1):
            pltpu.async_copy(TP_hbm.at[idx_v.at[r]], gb.at[r], sgs[r])

        def do_b(b, p):
            pltpu.make_async_copy(TP_hbm.at[idx_v.at[b]], gb.at[p],
                                  sgs[p]).wait()

            pn = (p + NR - 1) % NR

            @pl.when(b + NR - 1 < BPW)
            def _():
                pltpu.async_copy(TP_hbm.at[idx_v.at[b + NR - 1]], gb.at[pn],
                                 sgs[pn])

            @pl.when(b >= NR)
            def _():
                pltpu.make_async_copy(ob.at[p], out_hbm.at[b0 + b - NR],
                                      sos[p]).wait()

            Sb = [S_v[b, pl.ds(16 * j, 16)] for j in range(NJ)]
            gbp = gb.at[p]
            obp = ob.at[p]

            @plsc.parallel_loop(0, L, unroll=2)
            def row(l):
                x = [gbp[l, pl.ds(16 * j, 16)] + Sb[j] for j in range(NJ)]
                s = (((x[0] + x[1]) + (x[2] + x[3]))
                     + ((x[4] + x[5]) + (x[6] + x[7])))
                q = ((((x[0] * x[0] + x[1] * x[1])
                       + (x[2] * x[2] + x[3] * x[3]))
                      + ((x[4] * x[4] + x[5] * x[5])
                         + (x[6] * x[6] + x[7] * x[7]))))
                for perm in perms:
                    s = s + lane_swap(s, perm)
                    q = q + lane_swap(q, perm)
                mu = s * (1.0 / H)
                var = q * (1.0 / H) - mu * mu
                a = var + 1e-5
                ai = lax.bitcast_convert_type(a, jnp.int32)
                y = lax.bitcast_convert_type(
                    jnp.int32(0x5F375A86) - (ai >> 1), f32)
                y = y * (1.5 - 0.5 * a * y * y)
                for j in range(NJ):
                    obp[l, pl.ds(16 * j, 16)] = jnp.maximum(
                        (x[j] - mu) * y * gam[j] + bet[j], 0.0)
            pltpu.async_copy(obp, out_hbm.at[b0 + b], sos[p])

        def bodyn(i, carry):
            for r in range(NR):
                do_b(NR * i + r, r)
            return carry

        lax.fori_loop(0, BPW // NR, bodyn, 0)
        for r in range(NR):
            pltpu.make_async_copy(ob.at[r], out_hbm.at[b0 + BPW - NR + r],
                                  sos[r]).wait()

    return sc_main


def kernel(seq_tokens, state_vars, env_vars, aa_emb, pos_emb, pc_table, pc_W,
           pc_b, sW1, sb1, sW2, sb2, eW1, eb1, eW2, eb2, nW, nb, gamma, beta):
    B, L = seq_tokens.shape
    V, H = aa_emb.shape[0], nW.shape[1]
    f32 = jnp.float32

    TP, S, idx = pl.pallas_call(
        _prep_body,
        out_shape=[
            jax.ShapeDtypeStruct((L * V, H), f32),
            jax.ShapeDtypeStruct((B, H), f32),
            jax.ShapeDtypeStruct((B, L), jnp.int32),
        ],
    )(aa_emb, pos_emb, pc_table, pc_W, pc_b, seq_tokens, state_vars, env_vars,
      sW1, sb1, sW2, sb2, eW1, eb1, eW2, eb2, nW, nb)

    info = plsc.get_sparse_core_info()
    gam2 = gamma.reshape(1, -1)
    bet2 = beta.reshape(1, -1)

    sc_main = _make_sc_main(B, L, H, info.num_cores, info.num_subcores)
    out56 = sc_main(TP, idx, S, gam2, bet2)
    return out56[:, 0:L, :]


# speedup vs baseline: 1.3925x; 1.0494x over previous
import functools

import jax
import jax.numpy as jnp
from jax import lax
from jax.experimental import pallas as pl
from jax.experimental.pallas import tpu as pltpu
from jax.experimental.pallas import tpu_sc as plsc


def _prep_body(aa_ref, pos_ref, pc_ref, pcW_ref, pcb_ref, tok_ref, sv_ref,
               ev_ref, sW1_ref, sb1_ref, sW2_ref, sb2_ref, eW1_ref, eb1_ref,
               eW2_ref, eb2_ref, nW_ref, nb_ref, TP_out, S_out, idx_out):
    hp = jax.lax.Precision.HIGHEST
    L = idx_out.shape[1]
    nW = nW_ref[...]
    nW_aa, nW_pos, nW_pc = nW[0:32, :], nW[32:48, :], nW[48:64, :]
    nW_st, nW_ev = nW[64:96, :], nW[96:128, :]

    pc_feat = (jnp.dot(pc_ref[...], pcW_ref[...], precision=hp)
               + pcb_ref[...][None, :])
    T = (jnp.dot(aa_ref[...], nW_aa, precision=hp)
         + jnp.dot(pc_feat, nW_pc, precision=hp))
    P = jnp.dot(pos_ref[0:L, :], nW_pos, precision=hp)
    V, H = T.shape
    TP_out[...] = (P[:, None, :] + T[None, :, :]).reshape(L * V, H)

    tok = tok_ref[...]
    idx_out[...] = tok + V * jax.lax.broadcasted_iota(jnp.int32, tok.shape, 1)

    sv = sv_ref[...]
    f = jnp.concatenate([
        sv[:, 0:1] * 0.1,
        sv[:, 1:2] * (1.0 / 2000.0),
        jnp.log1p(jnp.maximum(sv[:, 2:3], 0.0)) * (1.0 / 20.0),
    ], axis=1)
    f = jnp.nan_to_num(f, nan=0.0, posinf=10.0, neginf=-10.0)
    hs = jnp.maximum(
        jnp.dot(f, sW1_ref[...], precision=hp) + sb1_ref[...][None, :], 0.0)
    s32 = jnp.dot(hs, sW2_ref[...], precision=hp) + sb2_ref[...][None, :]

    e = ev_ref[...] * 0.01
    e = jnp.nan_to_num(e, nan=0.0, posinf=10.0, neginf=-10.0)
    he = jnp.maximum(
        jnp.dot(e, eW1_ref[...], precision=hp) + eb1_ref[...][None, :], 0.0)
    e32 = jnp.dot(he, eW2_ref[...], precision=hp) + eb2_ref[...][None, :]

    S_out[...] = (jnp.dot(s32, nW_st, precision=hp)
                  + jnp.dot(e32, nW_ev, precision=hp) + nb_ref[...][None, :])


def _make_sc_main(BK, L, H, NC, NS):
    NW = NC * NS
    BPW = BK // NW
    NJ = H // 16
    f32 = jnp.float32

    NRG = 8
    NRO = 4

    @functools.partial(
        pl.kernel,
        out_type=jax.ShapeDtypeStruct((BK, L, H), f32),
        mesh=plsc.VectorSubcoreMesh(core_axis_name="c", subcore_axis_name="s"),
        scratch_types=[
            pltpu.VMEM((BPW, L), jnp.int32),
            pltpu.VMEM((BPW, H), f32),
            pltpu.VMEM((NRG, L, H), f32),
            pltpu.VMEM((NRO, L, H), f32),
            pltpu.VMEM((1, H), f32),
            pltpu.VMEM((1, H), f32),
        ] + [pltpu.SemaphoreType.DMA] * (NRG + NRO),
    )
    def sc_main(TP_hbm, idx_hbm, S_hbm, gam_hbm, bet_hbm, out_hbm,
                idx_v, S_v, gb, ob, gam_v, bet_v, *sems):
        sgs = sems[:NRG]
        sos = sems[NRG:]
        wid = lax.axis_index("s") * NC + lax.axis_index("c")
        b0 = wid * BPW
        pltpu.sync_copy(idx_hbm.at[pl.ds(b0, BPW)], idx_v)
        pltpu.sync_copy(S_hbm.at[pl.ds(b0, BPW)], S_v)
        pltpu.sync_copy(gam_hbm, gam_v)
        pltpu.sync_copy(bet_hbm, bet_v)

        gam = [gam_v[0, pl.ds(16 * j, 16)] for j in range(NJ)]
        bet = [bet_v[0, pl.ds(16 * j, 16)] for j in range(NJ)]
        lanes = lax.iota(jnp.int32, 16)
        perms = [(lanes ^ c)[:, None] for c in (8, 4, 2, 1)]
        dnums = lax.GatherDimensionNumbers(
            offset_dims=(), collapsed_slice_dims=(0,), start_index_map=(0,))

        def lane_swap(v, perm):
            return lax.gather(v, perm, dnums, slice_sizes=(1,),
                              mode=lax.GatherScatterMode.PROMISE_IN_BOUNDS)

        for r in range(NRG - 1):
            pltpu.async_copy(TP_hbm.at[idx_v.at[r]], gb.at[r], sgs[r])

        def do_b(b, p, po):
            pltpu.make_async_copy(TP_hbm.at[idx_v.at[b]], gb.at[p],
                                  sgs[p]).wait()

            pn = (p + NRG - 1) % NRG

            @pl.when(b + NRG - 1 < BPW)
            def _():
                pltpu.async_copy(TP_hbm.at[idx_v.at[b + NRG - 1]], gb.at[pn],
                                 sgs[pn])

            @pl.when(b >= NRO)
            def _():
                pltpu.make_async_copy(ob.at[po], out_hbm.at[b0 + b - NRO],
                                      sos[po]).wait()

            Sb = [S_v[b, pl.ds(16 * j, 16)] for j in range(NJ)]
            gbp = gb.at[p]
            obp = ob.at[po]

            @plsc.parallel_loop(0, L, unroll=2)
            def row(l):
                x = [gbp[l, pl.ds(16 * j, 16)] + Sb[j] for j in range(NJ)]
                s = (((x[0] + x[1]) + (x[2] + x[3]))
                     + ((x[4] + x[5]) + (x[6] + x[7])))
                q = ((((x[0] * x[0] + x[1] * x[1])
                       + (x[2] * x[2] + x[3] * x[3]))
                      + ((x[4] * x[4] + x[5] * x[5])
                         + (x[6] * x[6] + x[7] * x[7]))))
                for perm in perms:
                    s = s + lane_swap(s, perm)
                    q = q + lane_swap(q, perm)
                mu = s * (1.0 / H)
                var = q * (1.0 / H) - mu * mu
                a = var + 1e-5
                ai = lax.bitcast_convert_type(a, jnp.int32)
                y = lax.bitcast_convert_type(
                    jnp.int32(0x5F375A86) - (ai >> 1), f32)
                y = y * (1.5 - 0.5 * a * y * y)
                for j in range(NJ):
                    obp[l, pl.ds(16 * j, 16)] = jnp.maximum(
                        (x[j] - mu) * y * gam[j] + bet[j], 0.0)
            pltpu.async_copy(obp, out_hbm.at[b0 + b], sos[po])

        def bodyn(i, carry):
            for r in range(NRG):
                do_b(NRG * i + r, r, r % NRO)
            return carry

        lax.fori_loop(0, BPW // NRG, bodyn, 0)
        for r in range(NRO):
            pltpu.make_async_copy(ob.at[r], out_hbm.at[b0 + BPW - NRO + r],
                                  sos[r]).wait()

    return sc_main


def kernel(seq_tokens, state_vars, env_vars, aa_emb, pos_emb, pc_table, pc_W,
           pc_b, sW1, sb1, sW2, sb2, eW1, eb1, eW2, eb2, nW, nb, gamma, beta):
    B, L = seq_tokens.shape
    V, H = aa_emb.shape[0], nW.shape[1]
    f32 = jnp.float32

    TP, S, idx = pl.pallas_call(
        _prep_body,
        out_shape=[
            jax.ShapeDtypeStruct((L * V, H), f32),
            jax.ShapeDtypeStruct((B, H), f32),
            jax.ShapeDtypeStruct((B, L), jnp.int32),
        ],
    )(aa_emb, pos_emb, pc_table, pc_W, pc_b, seq_tokens, state_vars, env_vars,
      sW1, sb1, sW2, sb2, eW1, eb1, eW2, eb2, nW, nb)

    info = plsc.get_sparse_core_info()
    gam2 = gamma.reshape(1, -1)
    bet2 = beta.reshape(1, -1)

    sc_main = _make_sc_main(B, L, H, info.num_cores, info.num_subcores)
    out56 = sc_main(TP, idx, S, gam2, bet2)
    return out56[:, 0:L, :]


# speedup vs baseline: 1.4506x; 1.0417x over previous
import functools

import jax
import jax.numpy as jnp
from jax import lax
from jax.experimental import pallas as pl
from jax.experimental.pallas import tpu as pltpu
from jax.experimental.pallas import tpu_sc as plsc


def _prep_body(aa_ref, pos_ref, pc_ref, pcW_ref, pcb_ref, tok_ref, sv_ref,
               ev_ref, sW1_ref, sb1_ref, sW2_ref, sb2_ref, eW1_ref, eb1_ref,
               eW2_ref, eb2_ref, nW_ref, nb_ref, TP_out, S_out, idx_out):
    hp = jax.lax.Precision.HIGHEST
    L = idx_out.shape[1]
    nW = nW_ref[...]
    nW_aa, nW_pos, nW_pc = nW[0:32, :], nW[32:48, :], nW[48:64, :]
    nW_st, nW_ev = nW[64:96, :], nW[96:128, :]

    pc_feat = (jnp.dot(pc_ref[...], pcW_ref[...], precision=hp)
               + pcb_ref[...][None, :])
    T = (jnp.dot(aa_ref[...], nW_aa, precision=hp)
         + jnp.dot(pc_feat, nW_pc, precision=hp))
    P = jnp.dot(pos_ref[0:L, :], nW_pos, precision=hp)
    V, H = T.shape
    TP_out[...] = (P[:, None, :] + T[None, :, :]).reshape(L * V, H)

    tok = tok_ref[...]
    idx_out[...] = tok + V * jax.lax.broadcasted_iota(jnp.int32, tok.shape, 1)

    sv = sv_ref[...]
    f = jnp.concatenate([
        sv[:, 0:1] * 0.1,
        sv[:, 1:2] * (1.0 / 2000.0),
        jnp.log1p(jnp.maximum(sv[:, 2:3], 0.0)) * (1.0 / 20.0),
    ], axis=1)
    f = jnp.nan_to_num(f, nan=0.0, posinf=10.0, neginf=-10.0)
    hs = jnp.maximum(
        jnp.dot(f, sW1_ref[...], precision=hp) + sb1_ref[...][None, :], 0.0)
    s32 = jnp.dot(hs, sW2_ref[...], precision=hp) + sb2_ref[...][None, :]

    e = ev_ref[...] * 0.01
    e = jnp.nan_to_num(e, nan=0.0, posinf=10.0, neginf=-10.0)
    he = jnp.maximum(
        jnp.dot(e, eW1_ref[...], precision=hp) + eb1_ref[...][None, :], 0.0)
    e32 = jnp.dot(he, eW2_ref[...], precision=hp) + eb2_ref[...][None, :]

    S_out[...] = (jnp.dot(s32, nW_st, precision=hp)
                  + jnp.dot(e32, nW_ev, precision=hp) + nb_ref[...][None, :])


def _make_sc_main(BK, L, H, NC, NS, TPN):
    NW = NC * NS
    BPW = BK // NW
    NJ = H // 16
    f32 = jnp.float32

    NRG = 8
    NRO = 4

    @functools.partial(
        pl.kernel,
        out_type=jax.ShapeDtypeStruct((BK, L, H), f32),
        mesh=plsc.VectorSubcoreMesh(core_axis_name="c", subcore_axis_name="s"),
        scratch_types=[
            pltpu.VMEM((BPW, L), jnp.int32),
            pltpu.VMEM((BPW, H), f32),
            pltpu.VMEM((NRG, L, H), f32),
            pltpu.VMEM((NRO, L, H), f32),
            pltpu.VMEM((1, H), f32),
            pltpu.VMEM((1, H), f32),
            pltpu.VMEM_SHARED((TPN, H), f32),
        ] + [pltpu.SemaphoreType.DMA] * (NRG + NRO),
    )
    def sc_main(TP_hbm, idx_hbm, S_hbm, gam_hbm, bet_hbm, out_hbm,
                idx_v, S_v, gb, ob, gam_v, bet_v, tp_sh, *sems):
        sgs = sems[:NRG]
        sos = sems[NRG:]
        wid = lax.axis_index("s") * NC + lax.axis_index("c")
        b0 = wid * BPW
        pltpu.sync_copy(idx_hbm.at[pl.ds(b0, BPW)], idx_v)
        pltpu.sync_copy(S_hbm.at[pl.ds(b0, BPW)], S_v)
        pltpu.sync_copy(gam_hbm, gam_v)
        pltpu.sync_copy(bet_hbm, bet_v)

        @pl.when(lax.axis_index("s") == 0)
        def _():
            pltpu.sync_copy(TP_hbm, tp_sh)
        plsc.subcore_barrier()

        gam = [gam_v[0, pl.ds(16 * j, 16)] for j in range(NJ)]
        bet = [bet_v[0, pl.ds(16 * j, 16)] for j in range(NJ)]
        lanes = lax.iota(jnp.int32, 16)
        perms = [(lanes ^ c)[:, None] for c in (8, 4, 2, 1)]
        dnums = lax.GatherDimensionNumbers(
            offset_dims=(), collapsed_slice_dims=(0,), start_index_map=(0,))

        def lane_swap(v, perm):
            return lax.gather(v, perm, dnums, slice_sizes=(1,),
                              mode=lax.GatherScatterMode.PROMISE_IN_BOUNDS)

        for r in range(NRG - 1):
            pltpu.async_copy(tp_sh.at[idx_v.at[r]], gb.at[r], sgs[r])

        def do_b(b, p, po):
            pltpu.make_async_copy(tp_sh.at[idx_v.at[b]], gb.at[p],
                                  sgs[p]).wait()

            pn = (p + NRG - 1) % NRG

            @pl.when(b + NRG - 1 < BPW)
            def _():
                pltpu.async_copy(tp_sh.at[idx_v.at[b + NRG - 1]], gb.at[pn],
                                 sgs[pn])

            @pl.when(b >= NRO)
            def _():
                pltpu.make_async_copy(ob.at[po], out_hbm.at[b0 + b - NRO],
                                      sos[po]).wait()

            Sb = [S_v[b, pl.ds(16 * j, 16)] for j in range(NJ)]
            gbp = gb.at[p]
            obp = ob.at[po]

            @plsc.parallel_loop(0, L, unroll=2)
            def row(l):
                x = [gbp[l, pl.ds(16 * j, 16)] + Sb[j] for j in range(NJ)]
                s = (((x[0] + x[1]) + (x[2] + x[3]))
                     + ((x[4] + x[5]) + (x[6] + x[7])))
                q = ((((x[0] * x[0] + x[1] * x[1])
                       + (x[2] * x[2] + x[3] * x[3]))
                      + ((x[4] * x[4] + x[5] * x[5])
                         + (x[6] * x[6] + x[7] * x[7]))))
                for perm in perms:
                    s = s + lane_swap(s, perm)
                    q = q + lane_swap(q, perm)
                mu = s * (1.0 / H)
                var = q * (1.0 / H) - mu * mu
                a = var + 1e-5
                ai = lax.bitcast_convert_type(a, jnp.int32)
                y = lax.bitcast_convert_type(
                    jnp.int32(0x5F375A86) - (ai >> 1), f32)
                y = y * (1.5 - 0.5 * a * y * y)
                for j in range(NJ):
                    obp[l, pl.ds(16 * j, 16)] = jnp.maximum(
                        (x[j] - mu) * y * gam[j] + bet[j], 0.0)
            pltpu.async_copy(obp, out_hbm.at[b0 + b], sos[po])

        def bodyn(i, carry):
            for r in range(NRG):
                do_b(NRG * i + r, r, r % NRO)
            return carry

        lax.fori_loop(0, BPW // NRG, bodyn, 0)
        for r in range(NRO):
            pltpu.make_async_copy(ob.at[r], out_hbm.at[b0 + BPW - NRO + r],
                                  sos[r]).wait()

    return sc_main


def kernel(seq_tokens, state_vars, env_vars, aa_emb, pos_emb, pc_table, pc_W,
           pc_b, sW1, sb1, sW2, sb2, eW1, eb1, eW2, eb2, nW, nb, gamma, beta):
    B, L = seq_tokens.shape
    V, H = aa_emb.shape[0], nW.shape[1]
    f32 = jnp.float32

    TP, S, idx = pl.pallas_call(
        _prep_body,
        out_shape=[
            jax.ShapeDtypeStruct((L * V, H), f32),
            jax.ShapeDtypeStruct((B, H), f32),
            jax.ShapeDtypeStruct((B, L), jnp.int32),
        ],
    )(aa_emb, pos_emb, pc_table, pc_W, pc_b, seq_tokens, state_vars, env_vars,
      sW1, sb1, sW2, sb2, eW1, eb1, eW2, eb2, nW, nb)

    info = plsc.get_sparse_core_info()
    gam2 = gamma.reshape(1, -1)
    bet2 = beta.reshape(1, -1)

    sc_main = _make_sc_main(B, L, H, info.num_cores, info.num_subcores, L * V)
    return sc_main(TP, idx, S, gam2, bet2)
